# baseline (speedup 1.0000x reference)
import functools

import jax
import jax.numpy as jnp
from jax import lax
from jax.experimental import pallas as pl
from jax.experimental.pallas import tpu as pltpu
from jax.experimental.pallas import tpu_sc as plsc

N = 10000
NP = 10240
E = 320000
F_IN = 128
H = 128
C = 64
CK = 16000
BN = 1024


def _pack_pairs(yT):
    half = yT.shape[0] // 2
    yb = yT.astype(jnp.bfloat16)
    lo = lax.bitcast_convert_type(yb[:half], jnp.uint16).astype(jnp.uint32)
    hi = lax.bitcast_convert_type(yb[half:], jnp.uint16).astype(jnp.uint32)
    return lax.bitcast_convert_type((hi << 16) | lo, jnp.int32)


def _stage_a_body(x_ref, w_ref, out_ref):
    yT = lax.dot_general(w_ref[...], x_ref[...], (((0,), (1,)), ((), ())),
                         preferred_element_type=jnp.float32)
    out_ref[...] = _pack_pairs(yT)


def _stage_a(xp, W1):
    return pl.pallas_call(
        _stage_a_body,
        grid=(NP // BN,),
        in_specs=[pl.BlockSpec((BN, F_IN), lambda i: (i, 0)),
                  pl.BlockSpec((F_IN, H), lambda i: (0, 0))],
        out_specs=pl.BlockSpec((H // 2, BN), lambda i: (0, i)),
        out_shape=jax.ShapeDtypeStruct((H // 2, NP), jnp.int32),
    )(xp, W1)


def _stage_b_body(acc_ref, b1_ref, w2_ref, out_ref):
    a = jnp.maximum(acc_ref[...] + b1_ref[...], 0.0)
    yT = lax.dot_general(w2_ref[...], a, (((0,), (0,)), ((), ())),
                         preferred_element_type=jnp.float32)
    out_ref[...] = _pack_pairs(yT)


def _stage_b(acc1T, b1c, W2):
    return pl.pallas_call(
        _stage_b_body,
        grid=(NP // BN,),
        in_specs=[pl.BlockSpec((H, BN), lambda i: (0, i)),
                  pl.BlockSpec((H, 1), lambda i: (0, 0)),
                  pl.BlockSpec((H, C), lambda i: (0, 0))],
        out_specs=pl.BlockSpec((C // 2, BN), lambda i: (0, i)),
        out_shape=jax.ShapeDtypeStruct((C // 2, NP), jnp.int32),
    )(acc1T, b1c, W2)


def _stage_c_body(acc_ref, b2_ref, out_ref):
    z = acc_ref[...] + b2_ref[...]
    m = jnp.max(z, axis=0, keepdims=True)
    lse = jnp.log(jnp.sum(jnp.exp(z - m), axis=0, keepdims=True)) + m
    out_ref[...] = (z - lse).T


def _stage_c(acc2T, b2c):
    return pl.pallas_call(
        _stage_c_body,
        grid=(NP // BN,),
        in_specs=[pl.BlockSpec((C, BN), lambda i: (0, i)),
                  pl.BlockSpec((C, 1), lambda i: (0, 0))],
        out_specs=pl.BlockSpec((BN, C), lambda i: (i, 0)),
        out_shape=jax.ShapeDtypeStruct((N, C), jnp.float32),
    )(acc2T, b2c)


@functools.cache
def _make_spmm(F):
    FPT = F // 32
    PPT = FPT // 2
    info = plsc.get_sparse_core_info()
    nc = info.num_cores
    mesh = plsc.VectorSubcoreMesh(core_axis_name="c", subcore_axis_name="s")
    NCH = E // CK
    assert NCH % 2 == 0

    def body(hP, sdH, wH, outH, hrows, acc, sdb, wb, sem, hsem):
        fg = lax.axis_index("s") * nc + lax.axis_index("c")
        p0 = fg * PPT
        hcopy = pltpu.async_copy(hP.at[pl.ds(p0, PPT)], hrows, hsem)

        z16 = jnp.zeros((16,), jnp.float32)

        @plsc.parallel_loop(0, NP, 16, unroll=8)
        def zero_body(i):
            for f in range(FPT):
                acc[f, pl.ds(i, 16)] = z16

        hcopy.wait()

        def issue(ci, b):
            base = ci * CK
            pltpu.async_copy(sdH.at[pl.ds(base, CK)], sdb.at[b], sem.at[b])
            pltpu.async_copy(wH.at[pl.ds(base, CK)], wb.at[b], sem.at[b])

        def wait(b):
            pltpu.make_async_copy(sdH.at[pl.ds(0, CK)], sdb.at[b],
                                  sem.at[b]).wait()
            pltpu.make_async_copy(wH.at[pl.ds(0, CK)], wb.at[b],
                                  sem.at[b]).wait()

        issue(0, 0)
        himask = jnp.full((16,), -65536, jnp.int32)

        def pair_body(pr, _):
            for b in range(2):
                ci = pr * 2 + b

                @pl.when(ci + 1 < NCH)
                def _():
                    issue(ci + 1, 1 - b)

                wait(b)

                @plsc.parallel_loop(0, CK, 16, unroll=16)
                def group_body(o):
                    sd16 = sdb[b, pl.ds(o, 16)]
                    w16 = wb[b, pl.ds(o, 16)]
                    s16 = lax.bitwise_and(sd16, jnp.full((16,), 0xffff,
                                                         jnp.int32))
                    d16 = lax.shift_right_logical(sd16, 16)
                    for p in range(PPT):
                        p16 = jnp.full((16,), p, jnp.int32)
                        pk = plsc.load_gather(hrows, [p16, s16])
                        lo = plsc.bitcast(pk << 16, jnp.float32)
                        hi = plsc.bitcast(lax.bitwise_and(pk, himask),
                                          jnp.float32)
                        plsc.addupdate_scatter(
                            acc, [p16, d16], lo * w16)
                        plsc.addupdate_scatter(
                            acc, [jnp.full((16,), p + PPT, jnp.int32), d16],
                            hi * w16)

            return 0

        lax.fori_loop(0, NCH // 2, pair_body, 0)
        pltpu.sync_copy(acc.at[pl.ds(0, PPT)], outH.at[pl.ds(p0, PPT)])
        pltpu.sync_copy(acc.at[pl.ds(PPT, PPT)],
                        outH.at[pl.ds(F // 2 + p0, PPT)])

    return pl.kernel(
        body,
        out_type=jax.ShapeDtypeStruct((F, NP), jnp.float32),
        mesh=mesh,
        compiler_params=pltpu.CompilerParams(
            use_tc_tiling_on_sc=False, needs_layout_passes=False),
        scratch_types=[
            pltpu.VMEM((PPT, NP), jnp.int32),
            pltpu.VMEM((FPT, NP), jnp.float32),
            pltpu.VMEM((2, CK), jnp.int32),
            pltpu.VMEM((2, CK), jnp.float32),
            pltpu.SemaphoreType.DMA((2,)),
            pltpu.SemaphoreType.DMA,
        ],
    )


@jax.jit
def kernel(x, edge_index, edge_weight, W1, b1, W2, b2):
    src = edge_index[1]
    dst = edge_index[0]
    sd = jnp.bitwise_or(jnp.left_shift(dst, 16), src)
    h1P = _stage_a(x, W1)
    acc1T = _make_spmm(H)(h1P, sd, edge_weight)
    h2P = _stage_b(acc1T, b1.reshape(H, 1), W2)
    acc2T = _make_spmm(C)(h2P, sd, edge_weight)
    return _stage_c(acc2T, b2.reshape(C, 1))


# speedup vs baseline: 1.0931x; 1.0931x over previous
import functools

import jax
import jax.numpy as jnp
from jax import lax
from jax.experimental import pallas as pl
from jax.experimental.pallas import tpu as pltpu
from jax.experimental.pallas import tpu_sc as plsc

N = 10000
NP = 10240
E = 320000
F_IN = 128
H = 128
C = 64
CK = 16000
BN = 1024


def _pack_pairs(yT):
    half = yT.shape[0] // 2
    yb = yT.astype(jnp.bfloat16)
    lo = lax.bitcast_convert_type(yb[:half], jnp.uint16).astype(jnp.uint32)
    hi = lax.bitcast_convert_type(yb[half:], jnp.uint16).astype(jnp.uint32)
    return lax.bitcast_convert_type((hi << 16) | lo, jnp.int32)


def _stage_a_body(x_ref, w_ref, out_ref):
    yT = lax.dot_general(w_ref[...], x_ref[...], (((0,), (1,)), ((), ())),
                         preferred_element_type=jnp.float32)
    out_ref[...] = _pack_pairs(yT)


def _stage_a(xp, W1):
    return pl.pallas_call(
        _stage_a_body,
        grid=(NP // BN,),
        in_specs=[pl.BlockSpec((BN, F_IN), lambda i: (i, 0)),
                  pl.BlockSpec((F_IN, H), lambda i: (0, 0))],
        out_specs=pl.BlockSpec((H // 2, BN), lambda i: (0, i)),
        out_shape=jax.ShapeDtypeStruct((H // 2, NP), jnp.int32),
    )(xp, W1)


def _stage_b_body(acc_ref, b1_ref, w2_ref, out_ref):
    a = jnp.maximum(acc_ref[...] + b1_ref[...], 0.0)
    yT = lax.dot_general(w2_ref[...], a, (((0,), (0,)), ((), ())),
                         preferred_element_type=jnp.float32)
    out_ref[...] = _pack_pairs(yT)


def _stage_b(acc1T, b1c, W2):
    return pl.pallas_call(
        _stage_b_body,
        grid=(NP // BN,),
        in_specs=[pl.BlockSpec((H, BN), lambda i: (0, i)),
                  pl.BlockSpec((H, 1), lambda i: (0, 0)),
                  pl.BlockSpec((H, C), lambda i: (0, 0))],
        out_specs=pl.BlockSpec((C // 2, BN), lambda i: (0, i)),
        out_shape=jax.ShapeDtypeStruct((C // 2, NP), jnp.int32),
    )(acc1T, b1c, W2)


def _stage_c_body(acc_ref, b2_ref, out_ref):
    z = acc_ref[...] + b2_ref[...]
    m = jnp.max(z, axis=0, keepdims=True)
    lse = jnp.log(jnp.sum(jnp.exp(z - m), axis=0, keepdims=True)) + m
    out_ref[...] = (z - lse).T


def _stage_c(acc2T, b2c):
    return pl.pallas_call(
        _stage_c_body,
        grid=(NP // BN,),
        in_specs=[pl.BlockSpec((C, BN), lambda i: (0, i)),
                  pl.BlockSpec((C, 1), lambda i: (0, 0))],
        out_specs=pl.BlockSpec((BN, C), lambda i: (i, 0)),
        out_shape=jax.ShapeDtypeStruct((N, C), jnp.float32),
    )(acc2T, b2c)


@functools.cache
def _make_spmm(F):
    FPT = F // 32
    PPT = FPT // 2
    info = plsc.get_sparse_core_info()
    nc = info.num_cores
    mesh = plsc.VectorSubcoreMesh(core_axis_name="c", subcore_axis_name="s")
    NCH = E // CK
    assert NCH % 2 == 0

    def body(hP, sdH, wH, outH, hrows, acc, sdb, wb, sem, hsem):
        fg = lax.axis_index("s") * nc + lax.axis_index("c")
        p0 = fg * PPT
        hcopy = pltpu.async_copy(hP.at[pl.ds(p0, PPT)], hrows, hsem)

        z16 = jnp.zeros((16,), jnp.float32)

        @plsc.parallel_loop(0, NP, 16, unroll=8)
        def zero_body(i):
            for f in range(FPT):
                acc[f, pl.ds(i, 16)] = z16

        hcopy.wait()

        def issue(ci, b):
            base = ci * CK
            pltpu.async_copy(sdH.at[pl.ds(base, CK)], sdb.at[b], sem.at[b])
            pltpu.async_copy(wH.at[pl.ds(base, CK)], wb.at[b], sem.at[b])

        def wait(b):
            pltpu.make_async_copy(sdH.at[pl.ds(0, CK)], sdb.at[b],
                                  sem.at[b]).wait()
            pltpu.make_async_copy(wH.at[pl.ds(0, CK)], wb.at[b],
                                  sem.at[b]).wait()

        issue(0, 0)
        himask = jnp.full((16,), -65536, jnp.int32)

        def pair_body(pr, _):
            for b in range(2):
                ci = pr * 2 + b

                @pl.when(ci + 1 < NCH)
                def _():
                    issue(ci + 1, 1 - b)

                wait(b)

                @plsc.parallel_loop(0, CK, 16, unroll=8)
                def group_body(o):
                    sd16 = sdb[b, pl.ds(o, 16)]
                    w16 = wb[b, pl.ds(o, 16)]
                    s16 = lax.bitwise_and(sd16, jnp.full((16,), 0xffff,
                                                         jnp.int32))
                    d16 = lax.shift_right_logical(sd16, 16)
                    for p in range(PPT):
                        p16 = jnp.full((16,), p, jnp.int32)
                        pk = plsc.load_gather(hrows, [p16, s16])
                        lo = plsc.bitcast(pk << 16, jnp.float32)
                        hi = plsc.bitcast(lax.bitwise_and(pk, himask),
                                          jnp.float32)
                        plsc.addupdate_scatter(
                            acc, [p16, d16], lo * w16)
                        plsc.addupdate_scatter(
                            acc, [jnp.full((16,), p + PPT, jnp.int32), d16],
                            hi * w16)

            return 0

        lax.fori_loop(0, NCH // 2, pair_body, 0)
        pltpu.sync_copy(acc.at[pl.ds(0, PPT)], outH.at[pl.ds(p0, PPT)])
        pltpu.sync_copy(acc.at[pl.ds(PPT, PPT)],
                        outH.at[pl.ds(F // 2 + p0, PPT)])

    return pl.kernel(
        body,
        out_type=jax.ShapeDtypeStruct((F, NP), jnp.float32),
        mesh=mesh,
        compiler_params=pltpu.CompilerParams(
            use_tc_tiling_on_sc=False, needs_layout_passes=False),
        scratch_types=[
            pltpu.VMEM((PPT, NP), jnp.int32),
            pltpu.VMEM((FPT, NP), jnp.float32),
            pltpu.VMEM((2, CK), jnp.int32),
            pltpu.VMEM((2, CK), jnp.float32),
            pltpu.SemaphoreType.DMA((2,)),
            pltpu.SemaphoreType.DMA,
        ],
    )


@jax.jit
def kernel(x, edge_index, edge_weight, W1, b1, W2, b2):
    src = edge_index[1]
    dst = edge_index[0]
    sd = jnp.bitwise_or(jnp.left_shift(dst, 16), src)
    h1P = _stage_a(x, W1)
    acc1T = _make_spmm(H)(h1P, sd, edge_weight)
    h2P = _stage_b(acc1T, b1.reshape(H, 1), W2)
    acc2T = _make_spmm(C)(h2P, sd, edge_weight)
    return _stage_c(acc2T, b2.reshape(C, 1))


# speedup vs baseline: 1.1124x; 1.0177x over previous
import functools

import jax
import jax.numpy as jnp
from jax import lax
from jax.experimental import pallas as pl
from jax.experimental.pallas import tpu as pltpu
from jax.experimental.pallas import tpu_sc as plsc

N = 10000
NP = 10240
E = 320000
F_IN = 128
H = 128
C = 64
CK = 16000
BN = 1024


def _pack_pairs(yT):
    half = yT.shape[0] // 2
    yb = yT.astype(jnp.bfloat16)
    lo = lax.bitcast_convert_type(yb[:half], jnp.uint16).astype(jnp.uint32)
    hi = lax.bitcast_convert_type(yb[half:], jnp.uint16).astype(jnp.uint32)
    return lax.bitcast_convert_type((hi << 16) | lo, jnp.int32)


def _stage_a_body(x_ref, w_ref, out_ref):
    yT = lax.dot_general(w_ref[...], x_ref[...], (((0,), (1,)), ((), ())),
                         preferred_element_type=jnp.float32)
    out_ref[...] = _pack_pairs(yT)


def _stage_a(xp, W1):
    return pl.pallas_call(
        _stage_a_body,
        grid=(NP // BN,),
        in_specs=[pl.BlockSpec((BN, F_IN), lambda i: (i, 0)),
                  pl.BlockSpec((F_IN, H), lambda i: (0, 0))],
        out_specs=pl.BlockSpec((H // 2, BN), lambda i: (0, i)),
        out_shape=jax.ShapeDtypeStruct((H // 2, NP), jnp.int32),
    )(xp, W1)


def _stage_b_body(acc_ref, b1_ref, w2_ref, out_ref):
    a = jnp.maximum(acc_ref[...] + b1_ref[...], 0.0)
    yT = lax.dot_general(w2_ref[...], a, (((0,), (0,)), ((), ())),
                         preferred_element_type=jnp.float32)
    out_ref[...] = _pack_pairs(yT)


def _stage_b(acc1T, b1c, W2):
    return pl.pallas_call(
        _stage_b_body,
        grid=(NP // BN,),
        in_specs=[pl.BlockSpec((H, BN), lambda i: (0, i)),
                  pl.BlockSpec((H, 1), lambda i: (0, 0)),
                  pl.BlockSpec((H, C), lambda i: (0, 0))],
        out_specs=pl.BlockSpec((C // 2, BN), lambda i: (0, i)),
        out_shape=jax.ShapeDtypeStruct((C // 2, NP), jnp.int32),
    )(acc1T, b1c, W2)


def _stage_c_body(acc_ref, b2_ref, out_ref):
    z = acc_ref[...] + b2_ref[...]
    m = jnp.max(z, axis=0, keepdims=True)
    lse = jnp.log(jnp.sum(jnp.exp(z - m), axis=0, keepdims=True)) + m
    out_ref[...] = (z - lse).T


def _stage_c(acc2T, b2c):
    return pl.pallas_call(
        _stage_c_body,
        grid=(NP // BN,),
        in_specs=[pl.BlockSpec((C, BN), lambda i: (0, i)),
                  pl.BlockSpec((C, 1), lambda i: (0, 0))],
        out_specs=pl.BlockSpec((BN, C), lambda i: (i, 0)),
        out_shape=jax.ShapeDtypeStruct((N, C), jnp.float32),
    )(acc2T, b2c)


@functools.cache
def _make_spmm(F):
    FPT = F // 32
    PPT = FPT // 2
    info = plsc.get_sparse_core_info()
    nc = info.num_cores
    mesh = plsc.VectorSubcoreMesh(core_axis_name="c", subcore_axis_name="s")
    NCH = E // CK
    assert NCH % 2 == 0

    def body(hP, sdH, wH, outH, hrows, acc, sdb, wb, sem, hsem):
        fg = lax.axis_index("s") * nc + lax.axis_index("c")
        p0 = fg * PPT
        hcopy = pltpu.async_copy(hP.at[pl.ds(p0, PPT)], hrows, hsem)

        def issue(ci, b):
            base = ci * CK
            pltpu.async_copy(sdH.at[pl.ds(base, CK)], sdb.at[b], sem.at[b])
            pltpu.async_copy(wH.at[pl.ds(base, CK)], wb.at[b], sem.at[b])

        issue(0, 0)

        z16 = jnp.zeros((16,), jnp.float32)

        @plsc.parallel_loop(0, NP, 16, unroll=8)
        def zero_body(i):
            for f in range(FPT):
                acc[f, pl.ds(i, 16)] = z16

        hcopy.wait()

        def wait(b):
            pltpu.make_async_copy(sdH.at[pl.ds(0, CK)], sdb.at[b],
                                  sem.at[b]).wait()
            pltpu.make_async_copy(wH.at[pl.ds(0, CK)], wb.at[b],
                                  sem.at[b]).wait()

        himask = jnp.full((16,), -65536, jnp.int32)

        def pair_body(pr, _):
            for b in range(2):
                ci = pr * 2 + b

                @pl.when(ci + 1 < NCH)
                def _():
                    issue(ci + 1, 1 - b)

                wait(b)

                @plsc.parallel_loop(0, CK, 16, unroll=8)
                def group_body(o):
                    sd16 = sdb[b, pl.ds(o, 16)]
                    w16 = wb[b, pl.ds(o, 16)]
                    s16 = lax.bitwise_and(sd16, jnp.full((16,), 0xffff,
                                                         jnp.int32))
                    d16 = lax.shift_right_logical(sd16, 16)
                    for p in range(PPT):
                        p16 = jnp.full((16,), p, jnp.int32)
                        pk = plsc.load_gather(hrows, [p16, s16])
                        lo = plsc.bitcast(pk << 16, jnp.float32)
                        hi = plsc.bitcast(lax.bitwise_and(pk, himask),
                                          jnp.float32)
                        plsc.addupdate_scatter(
                            acc, [p16, d16], lo * w16)
                        plsc.addupdate_scatter(
                            acc, [jnp.full((16,), p + PPT, jnp.int32), d16],
                            hi * w16)

            return 0

        lax.fori_loop(0, NCH // 2, pair_body, 0)
        pltpu.sync_copy(acc.at[pl.ds(0, PPT)], outH.at[pl.ds(p0, PPT)])
        pltpu.sync_copy(acc.at[pl.ds(PPT, PPT)],
                        outH.at[pl.ds(F // 2 + p0, PPT)])

    return pl.kernel(
        body,
        out_type=jax.ShapeDtypeStruct((F, NP), jnp.float32),
        mesh=mesh,
        compiler_params=pltpu.CompilerParams(
            use_tc_tiling_on_sc=False, needs_layout_passes=False),
        scratch_types=[
            pltpu.VMEM((PPT, NP), jnp.int32),
            pltpu.VMEM((FPT, NP), jnp.float32),
            pltpu.VMEM((2, CK), jnp.int32),
            pltpu.VMEM((2, CK), jnp.float32),
            pltpu.SemaphoreType.DMA((2,)),
            pltpu.SemaphoreType.DMA,
        ],
    )


@jax.jit
def kernel(x, edge_index, edge_weight, W1, b1, W2, b2):
    src = edge_index[1]
    dst = edge_index[0]
    sd = jnp.bitwise_or(jnp.left_shift(dst, 16), src)
    h1P = _stage_a(x, W1)
    acc1T = _make_spmm(H)(h1P, sd, edge_weight)
    h2P = _stage_b(acc1T, b1.reshape(H, 1), W2)
    acc2T = _make_spmm(C)(h2P, sd, edge_weight)
    return _stage_c(acc2T, b2.reshape(C, 1))
